# baseline (speedup 1.0000x reference)
import functools

import jax
import jax.numpy as jnp
from jax import lax
from jax.experimental import pallas as pl
from jax.experimental.pallas import tpu as pltpu
from jax.experimental.pallas import tpu_sc as plsc

B = 16384
C = 26
V = 100000
D = 32

NC = 2
NS = 16
NW = NC * NS
L = 16
CB = 8192
HV = 50048
HV2 = V - HV

_mesh = plsc.VectorSubcoreMesh(core_axis_name="c", subcore_axis_name="s")


@functools.partial(
    pl.kernel,
    out_type=jax.ShapeDtypeStruct((C * D, B), jnp.float32),
    mesh=_mesh,
    compiler_params=pltpu.CompilerParams(needs_layout_passes=False),
    scratch_types=[
        pltpu.VMEM((HV,), jnp.float32),
        pltpu.VMEM((HV2,), jnp.float32),
        pltpu.VMEM((CB,), jnp.int32),
        pltpu.VMEM((CB,), jnp.float32),
        pltpu.VMEM((C * D,), jnp.float32),
        pltpu.SemaphoreType.DMA,
        pltpu.SemaphoreType.DMA,
    ],
)
def _embed(idx_hbm, tab_hbm, col_hbm, out_hbm,
           vec_a, vec_b, idx_v, o_v, col_v, sem_a, sem_b):
    w = lax.axis_index("s") * NC + lax.axis_index("c")

    def dma_a(r):
        return pltpu.make_async_copy(tab_hbm.at[r, pl.ds(0, HV)], vec_a, sem_a)

    def dma_b(r):
        return pltpu.make_async_copy(tab_hbm.at[r, pl.ds(HV, HV2)], vec_b, sem_b)

    pltpu.sync_copy(col_hbm, col_v)
    dma_a(w).start()
    dma_b(w).start()

    def per_c(c, carry):
        row = c * D + w
        nrow = jnp.minimum(c + 1, C - 1) * D + w
        bias = plsc.load_gather(col_v, [jnp.full((L,), row, jnp.int32)])

        def pass_a():
            @plsc.parallel_loop(0, CB // L, unroll=8)
            def _(i):
                ids = idx_v[pl.ds(i * L, L)]
                m = ids < HV
                o_v[pl.ds(i * L, L)] = plsc.load_gather(vec_a, [ids], mask=m) + bias

        def pass_b():
            @plsc.parallel_loop(0, CB // L, unroll=8)
            def _(i):
                s = pl.ds(i * L, L)
                ids = idx_v[s]
                m = ids >= HV
                g = plsc.load_gather(vec_b, [ids - HV], mask=m) + bias
                o_v[s] = jnp.where(m, g, o_v[s])

        dma_a(row).wait()
        pltpu.sync_copy(idx_hbm.at[c, pl.ds(0, CB)], idx_v)
        pass_a()
        dma_b(row).wait()
        pass_b()
        pltpu.sync_copy(o_v, out_hbm.at[row, pl.ds(0, CB)])
        pltpu.sync_copy(idx_hbm.at[c, pl.ds(CB, CB)], idx_v)
        pass_a()
        dma_a(nrow).start()
        pass_b()
        pltpu.sync_copy(o_v, out_hbm.at[row, pl.ds(CB, CB)])
        dma_b(nrow).start()
        return carry

    lax.fori_loop(0, C, per_c, 0)
    last = (C - 1) * D + w
    dma_a(last).wait()
    dma_b(last).wait()


def kernel(indices, tables, col_table):
    idx_t = indices.astype(jnp.int32).T
    tab_t = tables.transpose(0, 2, 1).reshape(C * D, V)
    out = _embed(idx_t, tab_t, col_table.reshape(C * D))
    return out.reshape(C, D, B).transpose(2, 0, 1)


# speedup vs baseline: 1.1466x; 1.1466x over previous
import functools

import jax
import jax.numpy as jnp
from jax import lax
from jax.experimental import pallas as pl
from jax.experimental.pallas import tpu as pltpu
from jax.experimental.pallas import tpu_sc as plsc

B = 16384
C = 26
V = 100000
D = 32

NC = 2
NS = 16
NW = NC * NS
L = 16
CB = 8192

_mesh = plsc.VectorSubcoreMesh(core_axis_name="c", subcore_axis_name="s")


@functools.partial(
    pl.kernel,
    out_type=jax.ShapeDtypeStruct((C * D, B), jnp.float32),
    mesh=_mesh,
    compiler_params=pltpu.CompilerParams(needs_layout_passes=False),
    scratch_types=[
        pltpu.VMEM((V,), jnp.float32),
        pltpu.VMEM((CB,), jnp.int32),
        pltpu.VMEM((CB,), jnp.float32),
        pltpu.VMEM((CB,), jnp.float32),
        pltpu.VMEM((C * D,), jnp.float32),
        pltpu.SemaphoreType.DMA,
        pltpu.SemaphoreType.DMA,
        pltpu.SemaphoreType.DMA,
    ],
)
def _embed(idx_hbm, tab_hbm, col_hbm, out_hbm,
           vec_v, idx_v, o0, o1, col_v, sem_v, sem_w0, sem_w1):
    w = lax.axis_index("s") * NC + lax.axis_index("c")

    def vdma(r):
        return pltpu.make_async_copy(tab_hbm.at[r], vec_v, sem_v)

    def wdma(o_ref, sem, r, hh):
        return pltpu.make_async_copy(
            o_ref, out_hbm.at[r, pl.ds(hh * CB, CB)], sem)

    pltpu.sync_copy(col_hbm, col_v)
    vdma(w).start()
    wdma(o0, sem_w0, w, 0).start()
    wdma(o1, sem_w1, w, 1).start()

    def per_c(c, carry):
        row = c * D + w
        nrow = jnp.minimum(c + 1, C - 1) * D + w
        bias = plsc.load_gather(col_v, [jnp.full((L,), row, jnp.int32)])

        pltpu.sync_copy(idx_hbm.at[c, pl.ds(0, CB)], idx_v)
        vdma(row).wait()
        wdma(o0, sem_w0, row, 0).wait()

        @plsc.parallel_loop(0, CB // L, unroll=8)
        def _g0(i):
            ids = idx_v[pl.ds(i * L, L)]
            o0[pl.ds(i * L, L)] = plsc.load_gather(vec_v, [ids]) + bias

        wdma(o0, sem_w0, row, 0).start()
        pltpu.sync_copy(idx_hbm.at[c, pl.ds(CB, CB)], idx_v)
        wdma(o1, sem_w1, row, 1).wait()

        @plsc.parallel_loop(0, CB // L, unroll=8)
        def _g1(i):
            ids = idx_v[pl.ds(i * L, L)]
            o1[pl.ds(i * L, L)] = plsc.load_gather(vec_v, [ids]) + bias

        vdma(nrow).start()
        wdma(o1, sem_w1, row, 1).start()
        return carry

    lax.fori_loop(0, C, per_c, 0)
    last = (C - 1) * D + w
    vdma(last).wait()
    wdma(o0, sem_w0, last, 0).wait()
    wdma(o1, sem_w1, last, 1).wait()


def kernel(indices, tables, col_table):
    idx_t = indices.astype(jnp.int32).T
    tab_t = tables.transpose(0, 2, 1).reshape(C * D, V)
    out = _embed(idx_t, tab_t, col_table.reshape(C * D))
    return out.reshape(C, D, B).transpose(2, 0, 1)
